# baseline (speedup 1.0000x reference)
import functools

import jax
import jax.numpy as jnp
from jax import lax
from jax.experimental import pallas as pl
from jax.experimental.pallas import tpu as pltpu
from jax.experimental.pallas import tpu_sc as plsc

N_NODES = 10000
N_EDGES = 320000
F_IN = 128
OUT_SCALE = 1.0 / 128.0

NW = 32
LANES = 128
PER_TILE = 10240
E_PAD = NW * PER_TILE
CHUNK = 512
SUB = CHUNK // LANES
N_CHUNKS = PER_TILE // CHUNK
TABLE_N = 10240
ZERO_PER_TILE = TABLE_N // 16


def _sc_edge_body(y_hbm, ei_hbm, ej_hbm, at_hbm, out_hbm,
                  eib, ejb, atb, eidx, cref, rows, msg, table,
                  semg, seme, sems):
    cid = lax.axis_index("c")
    sid = lax.axis_index("s")
    wid = cid * 16 + sid

    def zero_body(i, _):
        msg[0, i] = jnp.zeros((16,), jnp.float32)
        msg[1, i] = jnp.zeros((16,), jnp.float32)
        return 0
    lax.fori_loop(0, CHUNK, zero_body, 0)
    pltpu.sync_copy(msg.at[0], table.at[pl.ds(sid * ZERO_PER_TILE, CHUNK)])
    pltpu.sync_copy(msg.at[1, pl.ds(0, ZERO_PER_TILE - CHUNK)],
                    table.at[pl.ds(sid * ZERO_PER_TILE + CHUNK,
                                   ZERO_PER_TILE - CHUNK)])
    plsc.subcore_barrier()

    ebase = wid * PER_TILE
    lane = jnp.arange(16, dtype=jnp.int32)
    half_lane = lane // 2
    parity = lane & 1

    def load_edata(g):
        b = g % 2
        e0 = ebase + g * CHUNK
        return [
            pltpu.async_copy(ei_hbm.at[pl.ds(e0, CHUNK)], eib.at[b], seme),
            pltpu.async_copy(ej_hbm.at[pl.ds(e0, CHUNK)], ejb.at[b], seme),
            pltpu.async_copy(at_hbm.at[pl.ds(2 * e0, 2 * CHUNK)],
                             atb.at[b], seme),
        ]

    def fire_gathers(g):
        eb = ejb.at[g % 2]
        rb = rows.at[g % 2]
        return [pltpu.async_copy(y_hbm.at[eb.at[pl.ds(j * LANES, LANES)]],
                                 rb.at[pl.ds(j * LANES, LANES)], semg)
                for j in range(SUB)]

    def coef_compute(g):
        b = g % 2
        ab = atb.at[b]
        ev = eib.at[b]
        ew = ejb.at[b]
        ex = eidx.at[b]

        def cbody(k, _):
            v = ab[pl.ds(k * 16, 16)]
            t = (jnp.clip(v, -1.0, 1.0) + 1.0) * 0.5
            erow = k * 8 + half_lane
            plsc.store_scatter(cref, [erow, parity], t)
            return 0
        lax.fori_loop(0, 2 * CHUNK // 16, cbody, 0)

        def mbody(k, _):
            col = pl.ds(k * 16, 16)
            ei_v = ev[col]
            ej_v = ew[col]
            r = lax.shift_right_logical(k, 3)
            c2 = pl.ds((k & 7) * 16, 16)
            ex[r, c2] = jnp.where(ei_v != ej_v, ei_v, N_NODES)
            return 0
        lax.fori_loop(0, CHUNK // 16, mbody, 0)

    def edge_compute(g):
        mb = msg.at[g % 2]
        rb = rows.at[g % 2]

        @plsc.parallel_loop(0, CHUNK, 1, unroll=8)
        def edge_body(e):
            c = cref[e]
            tu = c[0]
            tv = c[1]
            s0 = rb[e, pl.ds(0, 16)]
            s1 = rb[e, pl.ds(16, 16)]
            s2 = rb[e, pl.ds(32, 16)]
            s3 = rb[e, pl.ds(48, 16)]
            h0 = s0 + tu * (s2 - s0)
            h1 = s1 + tu * (s3 - s1)
            mb[e] = h0 + tv * (h1 - h0)

    def fire_scatters(g):
        mb = msg.at[g % 2]
        ex = eidx.at[g % 2]
        return [pltpu.async_copy(mb.at[pl.ds(j * LANES, LANES)],
                                 table.at[ex.at[j]], sems, add=True)
                for j in range(SUB)]

    for cp in load_edata(0):
        cp.wait()
    gathers = fire_gathers(0)
    ed_next = load_edata(1)
    scatters = {}
    for g in range(N_CHUNKS):
        for cp in gathers:
            cp.wait()
        coef_compute(g)
        if g + 1 < N_CHUNKS:
            for cp in ed_next:
                cp.wait()
            gathers = fire_gathers(g + 1)
        if g + 2 < N_CHUNKS:
            ed_next = load_edata(g + 2)
        for cp in scatters.pop(g % 2, ()):
            cp.wait()
        edge_compute(g)
        scatters[g % 2] = fire_scatters(g)
    for cps in scatters.values():
        for cp in cps:
            cp.wait()

    plsc.subcore_barrier()

    pltpu.sync_copy(table.at[pl.ds(sid * ZERO_PER_TILE, ZERO_PER_TILE)],
                    out_hbm.at[cid, pl.ds(sid * ZERO_PER_TILE, ZERO_PER_TILE)])


_sc_edge = functools.partial(
    pl.kernel,
    out_type=jax.ShapeDtypeStruct((2, TABLE_N, 16), jnp.float32),
    mesh=plsc.VectorSubcoreMesh(core_axis_name="c", subcore_axis_name="s"),
    compiler_params=pltpu.CompilerParams(needs_layout_passes=False,
                                         use_tc_tiling_on_sc=False),
    scratch_types=[
        pltpu.VMEM((2, CHUNK), jnp.int32),
        pltpu.VMEM((2, CHUNK), jnp.int32),
        pltpu.VMEM((2, 2 * CHUNK), jnp.float32),
        pltpu.VMEM((2, SUB, LANES), jnp.int32),
        pltpu.VMEM((CHUNK, 16), jnp.float32),
        pltpu.VMEM((2, CHUNK, 64), jnp.float32),
        pltpu.VMEM((2, CHUNK, 16), jnp.float32),
        pltpu.VMEM_SHARED((TABLE_N, 16), jnp.float32),
        pltpu.SemaphoreType.DMA,
        pltpu.SemaphoreType.DMA,
        pltpu.SemaphoreType.DMA,
    ],
)(_sc_edge_body)


_GRID = 10
_BLK = N_NODES // _GRID


def _tc0_body(x_ref, w_ref, b_ref, y_ref, d_ref):
    r = (jnp.dot(x_ref[...], w_ref[...], preferred_element_type=jnp.float32)
         + b_ref[0:1, :])
    y_ref[...] = r[:, 0:64]
    d_ref[...] = r[:, 64:80]


def _tc_mid_body(d_in, part_ref, w_ref, b_ref, y_ref, d_ref):
    lin = jax.nn.relu(d_in[...])
    conv = jax.nn.relu(part_ref[0] + part_ref[1])
    r = (jnp.dot(lin, w_ref[0:16, :], preferred_element_type=jnp.float32)
         + jnp.dot(conv, w_ref[16:32, :], preferred_element_type=jnp.float32)
         + b_ref[0:1, :])
    y_ref[...] = r[:, 0:64]
    d_ref[...] = r[:, 64:80]


def _tc2_body(d_in, part_ref, w_ref, b_ref, y_ref, d_ref):
    ans = jax.nn.relu(d_in[...] + part_ref[0] + part_ref[1])
    r = (jnp.dot(ans, w_ref[...], preferred_element_type=jnp.float32)
         + b_ref[0:1, :])
    y_ref[...] = r[:, 0:64]
    d_ref[...] = r[:, 64:80]


def _tc3_body(d_in, part_ref, o_ref):
    o_ref[...] = (d_in[:, 0:2]
                  + part_ref[0, :, 0:2] + part_ref[1, :, 0:2]) * OUT_SCALE


def _rows_spec(w):
    return pl.BlockSpec((_BLK, w), lambda i: (i, 0))


def _full_spec(shp):
    return pl.BlockSpec(shp, lambda i: (0, 0))


_part_spec = pl.BlockSpec((2, _BLK, 16), lambda i: (0, i, 0))


def _call(body, in_specs, out_ws):
    return pl.pallas_call(
        body,
        grid=(_GRID,),
        in_specs=in_specs,
        out_specs=[pl.BlockSpec((_BLK, w), lambda i: (i, 0)) for w in out_ws],
        out_shape=[jax.ShapeDtypeStruct((N_NODES, w), jnp.float32)
                   for w in out_ws],
    )


def kernel(x, edge_index, edge_attr, Wc0, Wf0, bf0, Wc1, Wf1, bf1,
           Wc2, Wf2, bf2):
    f32 = jnp.float32

    def stack_conv(Wc, out_pad):
        blocks = [Wc[u, v] for u in range(2) for v in range(2)]
        if out_pad:
            blocks = [jnp.pad(b, ((0, 0), (0, out_pad))) for b in blocks]
        return jnp.concatenate(blocks, axis=1)

    w0 = jnp.concatenate([stack_conv(Wc0, 0), Wf0], axis=1)
    b0 = jnp.concatenate([jnp.zeros((64,), f32), bf0])
    w1 = jnp.concatenate([stack_conv(Wc1, 0), Wf1], axis=1)
    b1 = jnp.concatenate([jnp.zeros((64,), f32), bf1])
    w2 = jnp.concatenate([stack_conv(Wc2, 14),
                          jnp.pad(Wf2, ((0, 0), (0, 14)))], axis=1)
    b2 = jnp.concatenate([jnp.zeros((64,), f32), bf2,
                          jnp.zeros((14,), f32)])
    b0 = jnp.broadcast_to(b0, (8, 80))
    b1 = jnp.broadcast_to(b1, (8, 80))
    b2 = jnp.broadcast_to(b2, (8, 80))

    pad = E_PAD - N_EDGES
    pad_idx = jnp.arange(pad, dtype=jnp.int32) % N_NODES
    ei = jnp.concatenate([edge_index[0], pad_idx])
    ej = jnp.concatenate([edge_index[1], pad_idx])
    at = jnp.concatenate([edge_attr.reshape(2 * N_EDGES),
                          jnp.zeros((2 * pad,), f32)])

    y0, d0 = _call(_tc0_body,
                   [_rows_spec(F_IN), _full_spec((F_IN, 80)),
                    _full_spec((8, 80))], [64, 16])(x, w0, b0)
    part0 = _sc_edge(y0, ei, ej, at)

    y1, d1 = _call(_tc_mid_body,
                   [_rows_spec(16), _part_spec, _full_spec((32, 80)),
                    _full_spec((8, 80))], [64, 16])(d0, part0, w1, b1)
    part1 = _sc_edge(y1, ei, ej, at)

    y2, d2 = _call(_tc2_body,
                   [_rows_spec(16), _part_spec, _full_spec((16, 80)),
                    _full_spec((8, 80))], [64, 16])(d1, part1, w2, b2)
    part2 = _sc_edge(y2, ei, ej, at)

    (out,) = _call(_tc3_body, [_rows_spec(16), _part_spec], [2])(d2, part2)
    return out


# speedup vs baseline: 1.5378x; 1.5378x over previous
import functools

import jax
import jax.numpy as jnp
from jax import lax
from jax.experimental import pallas as pl
from jax.experimental.pallas import tpu as pltpu
from jax.experimental.pallas import tpu_sc as plsc

N_NODES = 10000
N_EDGES = 320000
F_IN = 128
OUT_SCALE = 1.0 / 128.0

NW = 32
LANES = 128
PER_TILE = 10240
E_PAD = NW * PER_TILE
CHUNK = 512
SUB = CHUNK // LANES
N_CHUNKS = PER_TILE // CHUNK
TABLE_N = 10240
ZERO_PER_TILE = TABLE_N // 16


def _sc_edge_body(y_hbm, ei_hbm, ej_hbm, ax_hbm, ay_hbm, out_hbm,
                  eib, ejb, axb, ayb, eidx, cref, rows, msg, table,
                  semg, seme, sems):
    cid = lax.axis_index("c")
    sid = lax.axis_index("s")
    wid = cid * 16 + sid

    def zero_body(i, _):
        msg[0, i] = jnp.zeros((16,), jnp.float32)
        msg[1, i] = jnp.zeros((16,), jnp.float32)
        return 0
    lax.fori_loop(0, CHUNK, zero_body, 0)
    pltpu.sync_copy(msg.at[0], table.at[pl.ds(sid * ZERO_PER_TILE, CHUNK)])
    pltpu.sync_copy(msg.at[1, pl.ds(0, ZERO_PER_TILE - CHUNK)],
                    table.at[pl.ds(sid * ZERO_PER_TILE + CHUNK,
                                   ZERO_PER_TILE - CHUNK)])
    plsc.subcore_barrier()

    rbase = wid * (PER_TILE // LANES)
    lane = jnp.arange(16, dtype=jnp.int32)
    zero16 = jnp.zeros((16,), jnp.int32)
    one16 = jnp.ones((16,), jnp.int32)

    def load_edata(g):
        b = g % 2
        r0 = rbase + g * SUB
        return [
            pltpu.async_copy(ei_hbm.at[pl.ds(r0, SUB)], eib.at[b], seme),
            pltpu.async_copy(ej_hbm.at[pl.ds(r0, SUB)], ejb.at[b], seme),
            pltpu.async_copy(ax_hbm.at[pl.ds(r0, SUB)], axb.at[b], seme),
            pltpu.async_copy(ay_hbm.at[pl.ds(r0, SUB)], ayb.at[b], seme),
        ]

    def fire_gathers(g):
        eb = ejb.at[g % 2]
        rb = rows.at[g % 2]
        return [pltpu.async_copy(y_hbm.at[eb.at[j]],
                                 rb.at[pl.ds(j * LANES, LANES)], semg)
                for j in range(SUB)]

    def coef_compute(g):
        b = g % 2
        av = axb.at[b]
        bv = ayb.at[b]
        ev = eib.at[b]
        ew = ejb.at[b]
        ex = eidx.at[b]

        def cbody(k, _):
            r = lax.shift_right_logical(k, 3)
            col = pl.ds((k & 7) * 16, 16)
            ta = (jnp.clip(av[r, col], -1.0, 1.0) + 1.0) * 0.5
            tb = (jnp.clip(bv[r, col], -1.0, 1.0) + 1.0) * 0.5
            erow = k * 16 + lane
            plsc.store_scatter(cref, [erow, zero16], ta)
            plsc.store_scatter(cref, [erow, one16], tb)
            ei_v = ev[r, col]
            ex[r, col] = jnp.where(ei_v != ew[r, col], ei_v, N_NODES)
            return 0
        lax.fori_loop(0, CHUNK // 16, cbody, 0)

    def edge_compute(g):
        mb = msg.at[g % 2]
        rb = rows.at[g % 2]

        @plsc.parallel_loop(0, CHUNK, 1, unroll=8)
        def edge_body(e):
            c = cref[e]
            tu = c[0]
            tv = c[1]
            s0 = rb[e, pl.ds(0, 16)]
            s1 = rb[e, pl.ds(16, 16)]
            s2 = rb[e, pl.ds(32, 16)]
            s3 = rb[e, pl.ds(48, 16)]
            h0 = s0 + tu * (s2 - s0)
            h1 = s1 + tu * (s3 - s1)
            mb[e] = h0 + tv * (h1 - h0)

    def fire_scatters(g):
        mb = msg.at[g % 2]
        ex = eidx.at[g % 2]
        return [pltpu.async_copy(mb.at[pl.ds(j * LANES, LANES)],
                                 table.at[ex.at[j]], sems, add=True)
                for j in range(SUB)]

    for cp in load_edata(0):
        cp.wait()
    gathers = fire_gathers(0)
    ed_next = load_edata(1)
    scatters = {}
    for g in range(N_CHUNKS):
        for cp in gathers:
            cp.wait()
        coef_compute(g)
        if g + 1 < N_CHUNKS:
            for cp in ed_next:
                cp.wait()
            gathers = fire_gathers(g + 1)
        if g + 2 < N_CHUNKS:
            ed_next = load_edata(g + 2)
        for cp in scatters.pop(g % 2, ()):
            cp.wait()
        edge_compute(g)
        scatters[g % 2] = fire_scatters(g)
    for cps in scatters.values():
        for cp in cps:
            cp.wait()

    plsc.subcore_barrier()

    pltpu.sync_copy(table.at[pl.ds(sid * ZERO_PER_TILE, ZERO_PER_TILE)],
                    out_hbm.at[cid, pl.ds(sid * ZERO_PER_TILE, ZERO_PER_TILE)])


_sc_edge = functools.partial(
    pl.kernel,
    out_type=jax.ShapeDtypeStruct((2, TABLE_N, 16), jnp.float32),
    mesh=plsc.VectorSubcoreMesh(core_axis_name="c", subcore_axis_name="s"),
    compiler_params=pltpu.CompilerParams(needs_layout_passes=False,
                                         use_tc_tiling_on_sc=False),
    scratch_types=[
        pltpu.VMEM((2, SUB, LANES), jnp.int32),
        pltpu.VMEM((2, SUB, LANES), jnp.int32),
        pltpu.VMEM((2, SUB, LANES), jnp.float32),
        pltpu.VMEM((2, SUB, LANES), jnp.float32),
        pltpu.VMEM((2, SUB, LANES), jnp.int32),
        pltpu.VMEM((CHUNK, 16), jnp.float32),
        pltpu.VMEM((2, CHUNK, 64), jnp.float32),
        pltpu.VMEM((2, CHUNK, 16), jnp.float32),
        pltpu.VMEM_SHARED((TABLE_N, 16), jnp.float32),
        pltpu.SemaphoreType.DMA,
        pltpu.SemaphoreType.DMA,
        pltpu.SemaphoreType.DMA,
    ],
)(_sc_edge_body)


_GRID = 10
_BLK = N_NODES // _GRID


def _tc0_body(x_ref, w_ref, b_ref, y_ref, d_ref):
    r = (jnp.dot(x_ref[...], w_ref[...], preferred_element_type=jnp.float32)
         + b_ref[0:1, :])
    y_ref[...] = r[:, 0:64]
    d_ref[...] = r[:, 64:80]


def _tc_mid_body(d_in, part_ref, w_ref, b_ref, y_ref, d_ref):
    lin = jax.nn.relu(d_in[...])
    conv = jax.nn.relu(part_ref[0] + part_ref[1])
    r = (jnp.dot(lin, w_ref[0:16, :], preferred_element_type=jnp.float32)
         + jnp.dot(conv, w_ref[16:32, :], preferred_element_type=jnp.float32)
         + b_ref[0:1, :])
    y_ref[...] = r[:, 0:64]
    d_ref[...] = r[:, 64:80]


def _tc2_body(d_in, part_ref, w_ref, b_ref, y_ref, d_ref):
    ans = jax.nn.relu(d_in[...] + part_ref[0] + part_ref[1])
    r = (jnp.dot(ans, w_ref[...], preferred_element_type=jnp.float32)
         + b_ref[0:1, :])
    y_ref[...] = r[:, 0:64]
    d_ref[...] = r[:, 64:80]


def _tc3_body(d_in, part_ref, o_ref):
    o_ref[...] = (d_in[:, 0:2]
                  + part_ref[0, :, 0:2] + part_ref[1, :, 0:2]) * OUT_SCALE


def _rows_spec(w):
    return pl.BlockSpec((_BLK, w), lambda i: (i, 0))


def _full_spec(shp):
    return pl.BlockSpec(shp, lambda i: (0, 0))


_part_spec = pl.BlockSpec((2, _BLK, 16), lambda i: (0, i, 0))


def _call(body, in_specs, out_ws):
    return pl.pallas_call(
        body,
        grid=(_GRID,),
        in_specs=in_specs,
        out_specs=[pl.BlockSpec((_BLK, w), lambda i: (i, 0)) for w in out_ws],
        out_shape=[jax.ShapeDtypeStruct((N_NODES, w), jnp.float32)
                   for w in out_ws],
    )


def kernel(x, edge_index, edge_attr, Wc0, Wf0, bf0, Wc1, Wf1, bf1,
           Wc2, Wf2, bf2):
    f32 = jnp.float32

    def stack_conv(Wc, out_pad):
        blocks = [Wc[u, v] for u in range(2) for v in range(2)]
        if out_pad:
            blocks = [jnp.pad(b, ((0, 0), (0, out_pad))) for b in blocks]
        return jnp.concatenate(blocks, axis=1)

    w0 = jnp.concatenate([stack_conv(Wc0, 0), Wf0], axis=1)
    b0 = jnp.concatenate([jnp.zeros((64,), f32), bf0])
    w1 = jnp.concatenate([stack_conv(Wc1, 0), Wf1], axis=1)
    b1 = jnp.concatenate([jnp.zeros((64,), f32), bf1])
    w2 = jnp.concatenate([stack_conv(Wc2, 14),
                          jnp.pad(Wf2, ((0, 0), (0, 14)))], axis=1)
    b2 = jnp.concatenate([jnp.zeros((64,), f32), bf2,
                          jnp.zeros((14,), f32)])
    b0 = jnp.broadcast_to(b0, (8, 80))
    b1 = jnp.broadcast_to(b1, (8, 80))
    b2 = jnp.broadcast_to(b2, (8, 80))

    pad = E_PAD - N_EDGES
    pad_idx = jnp.arange(pad, dtype=jnp.int32) % N_NODES

    def lanes128(v, fill):
        return jnp.concatenate([v, fill]).reshape(E_PAD // LANES, LANES)

    zpad = jnp.zeros((pad,), f32)
    ei = lanes128(edge_index[0], pad_idx)
    ej = lanes128(edge_index[1], pad_idx)
    ax = lanes128(edge_attr[:, 0], zpad)
    ay = lanes128(edge_attr[:, 1], zpad)

    y0, d0 = _call(_tc0_body,
                   [_rows_spec(F_IN), _full_spec((F_IN, 80)),
                    _full_spec((8, 80))], [64, 16])(x, w0, b0)
    part0 = _sc_edge(y0, ei, ej, ax, ay)

    y1, d1 = _call(_tc_mid_body,
                   [_rows_spec(16), _part_spec, _full_spec((32, 80)),
                    _full_spec((8, 80))], [64, 16])(d0, part0, w1, b1)
    part1 = _sc_edge(y1, ei, ej, ax, ay)

    y2, d2 = _call(_tc2_body,
                   [_rows_spec(16), _part_spec, _full_spec((16, 80)),
                    _full_spec((8, 80))], [64, 16])(d1, part1, w2, b2)
    part2 = _sc_edge(y2, ei, ej, ax, ay)

    (out,) = _call(_tc3_body, [_rows_spec(16), _part_spec], [2])(d2, part2)
    return out
